# baseline (speedup 1.0000x reference)
import functools

import jax
import jax.numpy as jnp
import numpy as np
from jax import lax
from jax.experimental import pallas as pl
from jax.experimental.pallas import tpu as pltpu
from jax.experimental.pallas import tpu_sc as plsc

NC = 2
NS = 16
NT = NC * NS
CH = 128
LANES = 16

_vector_mesh = plsc.VectorSubcoreMesh(
    core_axis_name="core", subcore_axis_name="subcore")

_P32 = np.stack([np.arange(16), 16 + np.arange(16)], axis=1).reshape(32)


def _col_swizzle(ncols):
  return np.concatenate([b * 32 + _P32 for b in range(ncols // 32)])


def _spmm_sc(table, src2, dst3, w2, n_pad, nbuf):
  feat = table.shape[1]
  per_tile = src2.shape[1]
  nch = per_tile // CH
  assert nch % (2 * nbuf) == 0
  rows_per_tile = n_pad // NS
  assert rows_per_tile % CH == 0

  dring = 2 * nbuf

  @functools.partial(
      pl.kernel,
      out_type=jax.ShapeDtypeStruct((NC, n_pad, feat), jnp.float32),
      mesh=_vector_mesh,
      scratch_types=[
          pltpu.VMEM((per_tile,), jnp.int32),
          pltpu.VMEM((2 * nbuf, CH), jnp.int32),
          [pltpu.VMEM((CH,), jnp.float32) for _ in range(nbuf)],
          [pltpu.VMEM((CH, feat), jnp.bfloat16) for _ in range(nbuf)],
          [pltpu.VMEM((CH, feat), jnp.float32) for _ in range(nbuf)],
          [pltpu.SemaphoreType.DMA for _ in range(nbuf)],
          [pltpu.SemaphoreType.DMA for _ in range(nbuf)],
          [pltpu.SemaphoreType.DMA for _ in range(nbuf)],
          [pltpu.SemaphoreType.DMA for _ in range(2 * nbuf)],
          pltpu.SemaphoreType.DMA,
          pltpu.VMEM_SHARED((n_pad, feat), jnp.float32),
      ],
      compiler_params=pltpu.CompilerParams(
          needs_layout_passes=False, use_tc_tiling_on_sc=False),
  )
  def k(table_hbm, src_hbm, dst_hbm, w_hbm, out_hbm,
        src_v, dst_v, wbufs, gbufs, sbufs, gsems, wsems, ssems, dsems,
        zsem, acc_sh):
    cid = lax.axis_index("core")
    sid = lax.axis_index("subcore")
    wid = cid * NS + sid

    def gather_start(c, i, di):
      pltpu.async_copy(
          table_hbm.at[src_v.at[pl.ds(c * CH, CH)]], gbufs[i], gsems[i])
      pltpu.async_copy(
          w_hbm.at[wid, pl.ds(c * CH, CH)], wbufs[i], wsems[i])
      pltpu.async_copy(dst_hbm.at[wid, c], dst_v.at[di], dsems[di])

    def gather_wait(c, i):
      pltpu.make_async_copy(
          table_hbm.at[src_v.at[pl.ds(c * CH, CH)]], gbufs[i],
          gsems[i]).wait()
      pltpu.make_async_copy(
          w_hbm.at[wid, pl.ds(c * CH, CH)], wbufs[i], wsems[i]).wait()

    def dst_wait(c, di):
      pltpu.make_async_copy(dst_hbm.at[wid, c], dst_v.at[di],
                            dsems[di]).wait()

    def scatter_start(c, i, di):
      pltpu.async_copy(sbufs[i], acc_sh.at[dst_v.at[di]], ssems[i],
                       add=True)

    def scatter_wait(c, i, di):
      pltpu.make_async_copy(sbufs[i], acc_sh.at[dst_v.at[di]],
                            ssems[i]).wait()

    pltpu.sync_copy(src_hbm.at[wid], src_v)

    @pl.loop(0, CH)
    def _(i):
      for f in range(feat // LANES):
        sbufs[0][i, pl.ds(f * LANES, LANES)] = jnp.zeros((LANES,), jnp.float32)

    row0 = sid * rows_per_tile
    nz = rows_per_tile // CH
    for r in range(nz):
      pltpu.async_copy(sbufs[0], acc_sh.at[pl.ds(row0 + r * CH, CH)], zsem)
    for r in range(nz):
      pltpu.make_async_copy(sbufs[0], acc_sh.at[pl.ds(row0 + r * CH, CH)],
                            zsem).wait()
    plsc.subcore_barrier()

    gather_start(0, 0, 0)
    gather_start(1, 1, 1)

    @pl.loop(0, nch, step=dring)
    def _(cb):
      for kk in range(dring):
        c = cb + kk
        i = kk % nbuf

        gather_wait(c, i)
        dst_wait(c, kk)

        @pl.loop(0, CH, unroll=2)
        def _(e):
          wv = plsc.load_gather(
              wbufs[i], [jnp.full((LANES,), e, jnp.int32)])
          for f in range(feat // 32):
            v = gbufs[i][e, pl.ds(f * 32, 32)]
            a, b = plsc.unpack(v, format=plsc.PackFormat.INTERLEAVED)
            sbufs[i][e, pl.ds(f * 32, LANES)] = a * wv
            sbufs[i][e, pl.ds(f * 32 + LANES, LANES)] = b * wv

        scatter_start(c, i, kk)

        j = (kk + 2) % nbuf
        cprev = c + 2 - nbuf

        @pl.when(cprev >= 0)
        def _():
          scatter_wait(cprev, j, (kk + 2 - nbuf) % dring)

        @pl.when(c + 2 < nch)
        def _():
          gather_start(c + 2, j, (kk + 2) % dring)

    for c in range(nch + 2 - nbuf, nch):
      scatter_wait(c, c % nbuf, c % dring)

    plsc.subcore_barrier()

    for r in range(nz):
      pltpu.async_copy(acc_sh.at[pl.ds(row0 + r * CH, CH)],
                       out_hbm.at[cid, pl.ds(row0 + r * CH, CH)], zsem)
    for r in range(nz):
      pltpu.make_async_copy(acc_sh.at[pl.ds(row0 + r * CH, CH)],
                            out_hbm.at[cid, pl.ds(row0 + r * CH, CH)],
                            zsem).wait()

  return k(table, src2, dst3, w2)


def _mid_body(p_ref, w1_ref, b1_ref, w2_ref, o_ref):
  n = o_ref.shape[0]
  ax = jnp.concatenate([p_ref[0, :n], p_ref[1, :n]], axis=1)
  h = jnp.dot(ax.astype(jnp.bfloat16), w1_ref[...],
              preferred_element_type=jnp.float32) + b1_ref[...]
  h = jnp.maximum(h, 0.0)
  o_ref[...] = jnp.dot(
      h.astype(jnp.bfloat16), w2_ref[...], preferred_element_type=jnp.float32
  ).astype(jnp.bfloat16)


def _out_body(q_ref, b2_ref, o_ref):
  n = o_ref.shape[0]
  z = q_ref[0, :n] + q_ref[1, :n] + b2_ref[...]
  m = jnp.max(z, axis=1, keepdims=True)
  s = jnp.sum(jnp.exp(z - m), axis=1, keepdims=True)
  o_ref[...] = (z - m) - jnp.log(s)


def _pad_edges(src, dst, w, groups, nbuf):
  e = w.shape[0]
  per = -(-(-(-e // groups)) // (CH * 2 * nbuf)) * (CH * 2 * nbuf)
  pad = groups * per - e
  return (jnp.pad(src, (0, pad)).reshape(groups, per),
          jnp.pad(dst, (0, pad)).reshape(groups, per // CH, CH),
          jnp.pad(w, (0, pad)).reshape(groups, per))


def kernel(x, edge_index, edge_weight, W1, b1, W2, b2):
  n_nodes, nfeat = x.shape
  nhid = W1.shape[1]
  nclass = W2.shape[1]
  n_pad = -(-n_nodes // (NS * CH)) * NS * CH

  src = edge_index[0].astype(jnp.int32)
  dst = edge_index[1].astype(jnp.int32)

  xs = x[:, _col_swizzle(nfeat)].astype(jnp.bfloat16)
  table1 = jnp.concatenate([xs[:, :nfeat // 2], xs[:, nfeat // 2:]], axis=0)
  W1b = W1.astype(jnp.bfloat16)
  W2s = W2[:, _col_swizzle(nclass)].astype(jnp.bfloat16)

  s1, d1, w1 = _pad_edges(src, dst, edge_weight, NS, 4)
  s1 = jnp.concatenate([s1, s1 + n_nodes], axis=0)
  d1 = jnp.concatenate([d1, d1], axis=0)
  w1 = jnp.concatenate([w1, w1], axis=0)

  s2, d2, w2e = _pad_edges(src, dst, edge_weight, NT, 4)

  p = _spmm_sc(table1, s1, d1, w1, n_pad, 4)

  support2 = pl.pallas_call(
      _mid_body,
      out_shape=jax.ShapeDtypeStruct((n_nodes, nclass), jnp.bfloat16),
  )(p, W1b, b1.reshape(1, nhid), W2s)

  q = _spmm_sc(support2, s2, d2, w2e, n_pad, 4)

  return pl.pallas_call(
      _out_body,
      out_shape=jax.ShapeDtypeStruct((n_nodes, nclass), jnp.float32),
  )(q, b2.reshape(1, nclass))


# speedup vs baseline: 1.0239x; 1.0239x over previous
import functools

import jax
import jax.numpy as jnp
import numpy as np
from jax import lax
from jax.experimental import pallas as pl
from jax.experimental.pallas import tpu as pltpu
from jax.experimental.pallas import tpu_sc as plsc

NC = 2
NS = 16
NT = NC * NS
CH = 128
LANES = 16

_vector_mesh = plsc.VectorSubcoreMesh(
    core_axis_name="core", subcore_axis_name="subcore")

_P32 = np.stack([np.arange(16), 16 + np.arange(16)], axis=1).reshape(32)


def _col_swizzle(ncols):
  return np.concatenate([b * 32 + _P32 for b in range(ncols // 32)])


def _col_unswizzle(ncols):
  return np.concatenate([b * 32 + np.argsort(_P32) for b in range(ncols // 32)])


def _spmm_sc(table, src2, dst3, w2, n_pad, nbuf):
  feat = table.shape[1]
  per_tile = src2.shape[1]
  nch = per_tile // CH
  assert nch % (2 * nbuf) == 0
  rows_per_tile = n_pad // NS
  assert rows_per_tile % CH == 0

  dring = 2 * nbuf

  @functools.partial(
      pl.kernel,
      out_type=jax.ShapeDtypeStruct((NC, n_pad, feat), jnp.float32),
      mesh=_vector_mesh,
      scratch_types=[
          pltpu.VMEM((per_tile,), jnp.int32),
          pltpu.VMEM((2 * nbuf, CH), jnp.int32),
          [pltpu.VMEM((CH,), jnp.float32) for _ in range(nbuf)],
          [pltpu.VMEM((CH, feat), jnp.bfloat16) for _ in range(nbuf)],
          [pltpu.VMEM((CH, feat), jnp.float32) for _ in range(nbuf)],
          [pltpu.SemaphoreType.DMA for _ in range(nbuf)],
          [pltpu.SemaphoreType.DMA for _ in range(nbuf)],
          [pltpu.SemaphoreType.DMA for _ in range(nbuf)],
          [pltpu.SemaphoreType.DMA for _ in range(2 * nbuf)],
          pltpu.SemaphoreType.DMA,
          pltpu.VMEM_SHARED((n_pad, feat), jnp.float32),
      ],
      compiler_params=pltpu.CompilerParams(
          needs_layout_passes=False, use_tc_tiling_on_sc=False),
  )
  def k(table_hbm, src_hbm, dst_hbm, w_hbm, out_hbm,
        src_v, dst_v, wbufs, gbufs, sbufs, gsems, wsems, ssems, dsems,
        zsem, acc_sh):
    cid = lax.axis_index("core")
    sid = lax.axis_index("subcore")
    wid = cid * NS + sid

    def gather_start(c, i, di):
      pltpu.async_copy(
          table_hbm.at[src_v.at[pl.ds(c * CH, CH)]], gbufs[i], gsems[i])
      pltpu.async_copy(
          w_hbm.at[wid, pl.ds(c * CH, CH)], wbufs[i], wsems[i])
      pltpu.async_copy(dst_hbm.at[wid, c], dst_v.at[di], dsems[di])

    def gather_wait(c, i):
      pltpu.make_async_copy(
          table_hbm.at[src_v.at[pl.ds(c * CH, CH)]], gbufs[i],
          gsems[i]).wait()
      pltpu.make_async_copy(
          w_hbm.at[wid, pl.ds(c * CH, CH)], wbufs[i], wsems[i]).wait()

    def dst_wait(c, di):
      pltpu.make_async_copy(dst_hbm.at[wid, c], dst_v.at[di],
                            dsems[di]).wait()

    def scatter_start(c, i, di):
      pltpu.async_copy(sbufs[i], acc_sh.at[dst_v.at[di]], ssems[i],
                       add=True)

    def scatter_wait(c, i, di):
      pltpu.make_async_copy(sbufs[i], acc_sh.at[dst_v.at[di]],
                            ssems[i]).wait()

    pltpu.sync_copy(src_hbm.at[wid], src_v)

    @pl.loop(0, CH)
    def _(i):
      for f in range(feat // LANES):
        sbufs[0][i, pl.ds(f * LANES, LANES)] = jnp.zeros((LANES,), jnp.float32)

    row0 = sid * rows_per_tile
    nz = rows_per_tile // CH
    for r in range(nz):
      pltpu.async_copy(sbufs[0], acc_sh.at[pl.ds(row0 + r * CH, CH)], zsem)
    for r in range(nz):
      pltpu.make_async_copy(sbufs[0], acc_sh.at[pl.ds(row0 + r * CH, CH)],
                            zsem).wait()
    plsc.subcore_barrier()

    gather_start(0, 0, 0)
    gather_start(1, 1, 1)

    @pl.loop(0, nch, step=dring)
    def _(cb):
      for kk in range(dring):
        c = cb + kk
        i = kk % nbuf

        gather_wait(c, i)
        dst_wait(c, kk)

        @pl.loop(0, CH, unroll=2)
        def _(e):
          wv = plsc.load_gather(
              wbufs[i], [jnp.full((LANES,), e, jnp.int32)])
          for f in range(feat // 32):
            v = gbufs[i][e, pl.ds(f * 32, 32)]
            a, b = plsc.unpack(v, format=plsc.PackFormat.INTERLEAVED)
            sbufs[i][e, pl.ds(f * 32, LANES)] = a * wv
            sbufs[i][e, pl.ds(f * 32 + LANES, LANES)] = b * wv

        scatter_start(c, i, kk)

        j = (kk + 2) % nbuf
        cprev = c + 2 - nbuf

        @pl.when(cprev >= 0)
        def _():
          scatter_wait(cprev, j, (kk + 2 - nbuf) % dring)

        @pl.when(c + 2 < nch)
        def _():
          gather_start(c + 2, j, (kk + 2) % dring)

    for c in range(nch + 2 - nbuf, nch):
      scatter_wait(c, c % nbuf, c % dring)

    plsc.subcore_barrier()

    for r in range(nz):
      pltpu.async_copy(acc_sh.at[pl.ds(row0 + r * CH, CH)],
                       out_hbm.at[cid, pl.ds(row0 + r * CH, CH)], zsem)
    for r in range(nz):
      pltpu.make_async_copy(acc_sh.at[pl.ds(row0 + r * CH, CH)],
                            out_hbm.at[cid, pl.ds(row0 + r * CH, CH)],
                            zsem).wait()

  return k(table, src2, dst3, w2)


def _mid_body(p_ref, w1_ref, b1_ref, w2_ref, o_ref):
  n = o_ref.shape[0]
  ax = jnp.concatenate([p_ref[0, :n], p_ref[1, :n]], axis=1)
  h = jnp.dot(ax.astype(jnp.bfloat16), w1_ref[...],
              preferred_element_type=jnp.float32) + b1_ref[...]
  h = jnp.maximum(h, 0.0)
  o_ref[...] = jnp.dot(
      h.astype(jnp.bfloat16), w2_ref[...], preferred_element_type=jnp.float32
  ).astype(jnp.bfloat16)


def _out_body(q_ref, b2_ref, o_ref):
  n = o_ref.shape[0]
  z = q_ref[0, :n] + q_ref[1, :n] + b2_ref[...]
  m = jnp.max(z, axis=1, keepdims=True)
  s = jnp.sum(jnp.exp(z - m), axis=1, keepdims=True)
  o_ref[...] = (z - m) - jnp.log(s)


def _pad_edges(src, dst, w, groups, nbuf):
  e = w.shape[0]
  per = -(-(-(-e // groups)) // (CH * 2 * nbuf)) * (CH * 2 * nbuf)
  pad = groups * per - e
  return (jnp.pad(src, (0, pad)).reshape(groups, per),
          jnp.pad(dst, (0, pad)).reshape(groups, per // CH, CH),
          jnp.pad(w, (0, pad)).reshape(groups, per))


def kernel(x, edge_index, edge_weight, W1, b1, W2, b2):
  n_nodes, nfeat = x.shape
  nhid = W1.shape[1]
  nclass = W2.shape[1]
  n_pad = -(-n_nodes // (NS * CH)) * NS * CH

  src = edge_index[0].astype(jnp.int32)
  dst = edge_index[1].astype(jnp.int32)

  xb = x.astype(jnp.bfloat16)
  table1 = jnp.concatenate([xb[:, :nfeat // 2], xb[:, nfeat // 2:]], axis=0)
  W1b = W1[_col_unswizzle(nfeat), :].astype(jnp.bfloat16)
  W2s = W2[:, _col_swizzle(nclass)].astype(jnp.bfloat16)

  s1, d1, w1 = _pad_edges(src, dst, edge_weight, NS, 4)
  s1 = jnp.concatenate([s1, s1 + n_nodes], axis=0)
  d1 = jnp.concatenate([d1, d1], axis=0)
  w1 = jnp.concatenate([w1, w1], axis=0)

  s2, d2, w2e = _pad_edges(src, dst, edge_weight, NT, 4)

  p = _spmm_sc(table1, s1, d1, w1, n_pad, 4)

  support2 = pl.pallas_call(
      _mid_body,
      out_shape=jax.ShapeDtypeStruct((n_nodes, nclass), jnp.bfloat16),
  )(p, W1b, b1.reshape(1, nhid), W2s)

  q = _spmm_sc(support2, s2, d2, w2e, n_pad, 4)

  return pl.pallas_call(
      _out_body,
      out_shape=jax.ShapeDtypeStruct((n_nodes, nclass), jnp.float32),
  )(q, b2.reshape(1, nclass))


# speedup vs baseline: 1.0483x; 1.0238x over previous
import functools

import jax
import jax.numpy as jnp
import numpy as np
from jax import lax
from jax.experimental import pallas as pl
from jax.experimental.pallas import tpu as pltpu
from jax.experimental.pallas import tpu_sc as plsc

NC = 2
NS = 16
NT = NC * NS
CH = 128
LANES = 16

_vector_mesh = plsc.VectorSubcoreMesh(
    core_axis_name="core", subcore_axis_name="subcore")

_P32 = np.stack([np.arange(16), 16 + np.arange(16)], axis=1).reshape(32)


def _col_swizzle(ncols):
  return np.concatenate([b * 32 + _P32 for b in range(ncols // 32)])


def _spmm_sc(table, src2, dst3, w2, n_pad, nbuf):
  feat = table.shape[1]
  per_tile = src2.shape[1]
  nch = per_tile // CH
  assert nch % (2 * nbuf) == 0
  rows_per_tile = n_pad // NS
  assert rows_per_tile % CH == 0

  dring = 2 * nbuf

  @functools.partial(
      pl.kernel,
      out_type=jax.ShapeDtypeStruct((NC, n_pad, feat), jnp.float32),
      mesh=_vector_mesh,
      scratch_types=[
          pltpu.VMEM((per_tile,), jnp.int32),
          pltpu.VMEM((2 * nbuf, CH), jnp.int32),
          [pltpu.VMEM((CH,), jnp.float32) for _ in range(nbuf)],
          [pltpu.VMEM((CH, feat), jnp.bfloat16) for _ in range(nbuf)],
          [pltpu.VMEM((CH, feat), jnp.float32) for _ in range(nbuf)],
          [pltpu.SemaphoreType.DMA for _ in range(nbuf)],
          [pltpu.SemaphoreType.DMA for _ in range(nbuf)],
          [pltpu.SemaphoreType.DMA for _ in range(nbuf)],
          [pltpu.SemaphoreType.DMA for _ in range(2 * nbuf)],
          pltpu.SemaphoreType.DMA,
          pltpu.VMEM_SHARED((n_pad, feat), jnp.float32),
      ],
      compiler_params=pltpu.CompilerParams(
          needs_layout_passes=False, use_tc_tiling_on_sc=False),
  )
  def k(table_hbm, src_hbm, dst_hbm, w_hbm, out_hbm,
        src_v, dst_v, wbufs, gbufs, sbufs, gsems, wsems, ssems, dsems,
        zsem, acc_sh):
    cid = lax.axis_index("core")
    sid = lax.axis_index("subcore")
    wid = cid * NS + sid

    def gather_start(c, i, di):
      pltpu.async_copy(
          table_hbm.at[src_v.at[pl.ds(c * CH, CH)]], gbufs[i], gsems[i])
      pltpu.async_copy(
          w_hbm.at[wid, pl.ds(c * CH, CH)], wbufs[i], wsems[i])
      pltpu.async_copy(dst_hbm.at[wid, c], dst_v.at[di], dsems[di])

    def gather_wait(c, i):
      pltpu.make_async_copy(
          table_hbm.at[src_v.at[pl.ds(c * CH, CH)]], gbufs[i],
          gsems[i]).wait()
      pltpu.make_async_copy(
          w_hbm.at[wid, pl.ds(c * CH, CH)], wbufs[i], wsems[i]).wait()

    def dst_wait(c, di):
      pltpu.make_async_copy(dst_hbm.at[wid, c], dst_v.at[di],
                            dsems[di]).wait()

    def scatter_start(c, i, di):
      pltpu.async_copy(sbufs[i], acc_sh.at[dst_v.at[di]], ssems[i],
                       add=True)

    def scatter_wait(c, i, di):
      pltpu.make_async_copy(sbufs[i], acc_sh.at[dst_v.at[di]],
                            ssems[i]).wait()

    pltpu.sync_copy(src_hbm.at[wid], src_v)

    @pl.loop(0, CH)
    def _(i):
      for f in range(feat // LANES):
        sbufs[0][i, pl.ds(f * LANES, LANES)] = jnp.zeros((LANES,), jnp.float32)

    row0 = sid * rows_per_tile
    nz = rows_per_tile // CH
    for r in range(nz):
      pltpu.async_copy(sbufs[0], acc_sh.at[pl.ds(row0 + r * CH, CH)], zsem)
    for r in range(nz):
      pltpu.make_async_copy(sbufs[0], acc_sh.at[pl.ds(row0 + r * CH, CH)],
                            zsem).wait()
    plsc.subcore_barrier()

    gather_start(0, 0, 0)
    gather_start(1, 1, 1)

    @pl.loop(0, nch, step=dring)
    def _(cb):
      for kk in range(dring):
        c = cb + kk
        i = kk % nbuf

        gather_wait(c, i)
        dst_wait(c, kk)

        @pl.loop(0, CH, unroll=2)
        def _(e):
          wv = plsc.load_gather(
              wbufs[i], [jnp.full((LANES,), e, jnp.int32)])
          for f in range(feat // 32):
            v = gbufs[i][e, pl.ds(f * 32, 32)]
            a, b = plsc.unpack(v, format=plsc.PackFormat.INTERLEAVED)
            sbufs[i][e, pl.ds(f * 32, LANES)] = a * wv
            sbufs[i][e, pl.ds(f * 32 + LANES, LANES)] = b * wv

        scatter_start(c, i, kk)

        j = (kk + 2) % nbuf
        cprev = c + 2 - nbuf

        @pl.when(cprev >= 0)
        def _():
          scatter_wait(cprev, j, (kk + 2 - nbuf) % dring)

        @pl.when(c + 2 < nch)
        def _():
          gather_start(c + 2, j, (kk + 2) % dring)

    for c in range(nch + 2 - nbuf, nch):
      scatter_wait(c, c % nbuf, c % dring)

    plsc.subcore_barrier()

    for r in range(nz):
      pltpu.async_copy(acc_sh.at[pl.ds(row0 + r * CH, CH)],
                       out_hbm.at[cid, pl.ds(row0 + r * CH, CH)], zsem)
    for r in range(nz):
      pltpu.make_async_copy(acc_sh.at[pl.ds(row0 + r * CH, CH)],
                            out_hbm.at[cid, pl.ds(row0 + r * CH, CH)],
                            zsem).wait()

  return k(table, src2, dst3, w2)


def _mm1_body(x_ref, w_ref, o_ref):
  s = jnp.dot(x_ref[...].astype(jnp.bfloat16), w_ref[...],
              preferred_element_type=jnp.float32)
  o_ref[0] = s[:, :64].astype(jnp.bfloat16)
  o_ref[1] = s[:, 64:].astype(jnp.bfloat16)


def _l2_body(p_ref, b1_ref, w2_ref, o_ref):
  n = o_ref.shape[0]
  h = jnp.concatenate([p_ref[0, :n], p_ref[1, :n]], axis=1) + b1_ref[...]
  h = jnp.maximum(h, 0.0)
  o_ref[...] = jnp.dot(
      h.astype(jnp.bfloat16), w2_ref[...], preferred_element_type=jnp.float32
  ).astype(jnp.bfloat16)


def _out_body(q_ref, b2_ref, o_ref):
  n = o_ref.shape[0]
  z = q_ref[0, :n] + q_ref[1, :n] + b2_ref[...]
  m = jnp.max(z, axis=1, keepdims=True)
  s = jnp.sum(jnp.exp(z - m), axis=1, keepdims=True)
  o_ref[...] = (z - m) - jnp.log(s)


def _pad_edges(src, dst, w, groups, nbuf):
  e = w.shape[0]
  per = -(-(-(-e // groups)) // (CH * 2 * nbuf)) * (CH * 2 * nbuf)
  pad = groups * per - e
  return (jnp.pad(src, (0, pad)).reshape(groups, per),
          jnp.pad(dst, (0, pad)).reshape(groups, per // CH, CH),
          jnp.pad(w, (0, pad)).reshape(groups, per))


def kernel(x, edge_index, edge_weight, W1, b1, W2, b2):
  n_nodes, nfeat = x.shape
  nhid = W1.shape[1]
  nclass = W2.shape[1]
  n_pad = -(-n_nodes // (NS * CH)) * NS * CH

  src = edge_index[0].astype(jnp.int32)
  dst = edge_index[1].astype(jnp.int32)

  W1s = W1[:, _col_swizzle(nhid)].astype(jnp.bfloat16)
  W2s = W2[:, _col_swizzle(nclass)].astype(jnp.bfloat16)

  s1, d1, w1 = _pad_edges(src, dst, edge_weight, NS, 4)
  s1 = jnp.concatenate([s1, s1 + n_nodes], axis=0)
  d1 = jnp.concatenate([d1, d1], axis=0)
  w1 = jnp.concatenate([w1, w1], axis=0)

  s2, d2, w2e = _pad_edges(src, dst, edge_weight, NT, 4)

  support = pl.pallas_call(
      _mm1_body,
      out_shape=jax.ShapeDtypeStruct((2, n_nodes, nhid // 2), jnp.bfloat16),
  )(x, W1s)

  p = _spmm_sc(support.reshape(2 * n_nodes, nhid // 2), s1, d1, w1, n_pad, 4)

  support2 = pl.pallas_call(
      _l2_body,
      out_shape=jax.ShapeDtypeStruct((n_nodes, nclass), jnp.bfloat16),
  )(p, b1.reshape(1, nhid), W2s)

  q = _spmm_sc(support2, s2, d2, w2e, n_pad, 4)

  return pl.pallas_call(
      _out_body,
      out_shape=jax.ShapeDtypeStruct((n_nodes, nclass), jnp.float32),
  )(q, b2.reshape(1, nclass))
